# baseline (speedup 1.0000x reference)
import functools

import jax
import jax.numpy as jnp
from jax import lax
from jax.experimental import pallas as pl
from jax.experimental.pallas import tpu as pltpu
from jax.experimental.pallas import tpu_sc as plsc

D_MODEL = 64
T_SEQ = 200
LANES = 16
NBUF = 2


@functools.lru_cache(maxsize=None)
def _make(batch):
    info = plsc.get_sparse_core_info()
    nc, ns = info.num_cores, info.num_subcores
    nw = nc * ns
    bw = batch // nw
    ngrp = bw // LANES

    mesh = plsc.VectorSubcoreMesh(core_axis_name="c", subcore_axis_name="s")

    @functools.partial(
        pl.kernel,
        mesh=mesh,
        out_type=jax.ShapeDtypeStruct((T_SEQ, D_MODEL, batch), jnp.float32),
        scratch_types=[
            pltpu.VMEM((T_SEQ, bw), jnp.int32),
            pltpu.VMEM((NBUF, bw), jnp.int32),
            pltpu.VMEM((NBUF, bw), jnp.int32),
            pltpu.VMEM((NBUF, bw, 128), jnp.float32),
            pltpu.VMEM((D_MODEL, bw), jnp.float32),
            pltpu.SemaphoreType.DMA,
        ],
        compiler_params=pltpu.CompilerParams(use_tc_tiling_on_sc=True,
                                             needs_layout_passes=False),
    )
    def gather_kernel(xt_hbm, tab_hbm, out_hbm, idx_v, gidx_v, par_v,
                      rows_v, blk_v, sem):
        wid = lax.axis_index("s") * nc + lax.axis_index("c")
        b0 = wid * bw
        pltpu.sync_copy(xt_hbm.at[:, pl.ds(b0, bw)], idx_v)

        def fire(b, t):
            for j in range(ngrp):
                sl = pl.ds(j * LANES, LANES)
                r = idx_v[t, sl]
                gidx_v[b, sl] = lax.shift_right_logical(r, 1)
                par_v[b, sl] = lax.shift_left(
                    lax.bitwise_and(r, jnp.int32(1)), 6)
            pltpu.async_copy(tab_hbm.at[gidx_v.at[b]], rows_v.at[b], sem)

        def drain(b):
            pltpu.make_async_copy(
                tab_hbm.at[gidx_v.at[b]], rows_v.at[b], sem).wait()

        def emit(b, t):
            rowid = [lax.iota(jnp.int32, LANES) + j * LANES
                     for j in range(ngrp)]
            par = [par_v[b, pl.ds(j * LANES, LANES)] for j in range(ngrp)]
            for c in range(D_MODEL):
                for j in range(ngrp):
                    blk_v[c, pl.ds(j * LANES, LANES)] = plsc.load_gather(
                        rows_v.at[b], [rowid[j], par[j] + c])
            pltpu.sync_copy(blk_v, out_hbm.at[t, :, pl.ds(b0, bw)])

        fire(0, 0)

        def body(i, carry):
            t0 = i * NBUF

            @pl.when(t0 + 1 < T_SEQ)
            def _():
                fire(1, t0 + 1)

            drain(0)
            emit(0, t0)

            @pl.when(t0 + 2 < T_SEQ)
            def _():
                fire(0, t0 + 2)

            @pl.when(t0 + 1 < T_SEQ)
            def _():
                drain(1)
                emit(1, t0 + 1)

            return carry

        lax.fori_loop(0, T_SEQ // NBUF, body, 0)

    return gather_kernel


def kernel(x, table):
    b, t = x.shape
    xt = x.T.astype(jnp.int32)
    tab2 = table.reshape(table.shape[0] // 2, 128)
    o = _make(b)(xt, tab2)
    return jnp.transpose(o, (2, 0, 1))


# speedup vs baseline: 1.6219x; 1.6219x over previous
import functools

import jax
import jax.numpy as jnp
from jax import lax
from jax.experimental import pallas as pl
from jax.experimental.pallas import tpu as pltpu
from jax.experimental.pallas import tpu_sc as plsc

D_MODEL = 64
T_SEQ = 200
BROWS = 4
NBUF = 2
STREAMS = [(0, 128), (128, 128), (256, 128), (384, 128), (512, 128),
           (640, 128), (768, 32)]


@functools.lru_cache(maxsize=None)
def _make(batch):
    info = plsc.get_sparse_core_info()
    nc, ns = info.num_cores, info.num_subcores
    nw = nc * ns
    rows_w = batch // nw
    idx_w = rows_w * T_SEQ
    n_chunks = rows_w // BROWS
    outer = n_chunks // NBUF
    ch_rows = BROWS * T_SEQ

    mesh = plsc.VectorSubcoreMesh(core_axis_name="c", subcore_axis_name="s")

    @functools.partial(
        pl.kernel,
        mesh=mesh,
        out_type=jax.ShapeDtypeStruct((batch, T_SEQ, D_MODEL), jnp.float32),
        scratch_types=[
            pltpu.VMEM((idx_w,), jnp.int32),
            pltpu.VMEM((NBUF, ch_rows, D_MODEL), jnp.float32),
            pltpu.SemaphoreType.DMA,
        ],
        compiler_params=pltpu.CompilerParams(use_tc_tiling_on_sc=False),
    )
    def gather_kernel(idx_hbm, table_hbm, out_hbm, idx_v, rows_v, sem):
        wid = lax.axis_index("s") * nc + lax.axis_index("c")
        base_row = wid * rows_w
        pltpu.sync_copy(idx_hbm.at[pl.ds(wid * idx_w, idx_w)], idx_v)

        def fire(b, chunk):
            for off, sz in STREAMS:
                pltpu.async_copy(
                    table_hbm.at[idx_v.at[pl.ds(chunk * ch_rows + off, sz)]],
                    rows_v.at[b, pl.ds(off, sz)], sem)

        def drain(b):
            for off, sz in STREAMS:
                pltpu.make_async_copy(
                    table_hbm.at[idx_v.at[pl.ds(off, sz)]],
                    rows_v.at[b, pl.ds(off, sz)], sem).wait()

        def write(b, chunk):
            for k in range(BROWS):
                pltpu.sync_copy(
                    rows_v.at[b, pl.ds(k * T_SEQ, T_SEQ)],
                    out_hbm.at[base_row + chunk * BROWS + k])

        fire(0, 0)

        def body(i, carry):
            c0 = i * NBUF
            fire(1, c0 + 1)
            drain(0)
            write(0, c0)

            @pl.when(i < outer - 1)
            def _():
                fire(0, c0 + 2)

            drain(1)
            write(1, c0 + 1)
            return carry

        lax.fori_loop(0, outer, body, 0)

    return gather_kernel


def kernel(x, table):
    b, t = x.shape
    idx = x.reshape(b * t).astype(jnp.int32)
    return _make(b)(idx, table)


# speedup vs baseline: 1.6925x; 1.0435x over previous
import functools

import jax
import jax.numpy as jnp
from jax import lax
from jax.experimental import pallas as pl
from jax.experimental.pallas import tpu as pltpu
from jax.experimental.pallas import tpu_sc as plsc

D_MODEL = 64
DP = 128
T_SEQ = 200
W_T = 2048
BROWS = 2
NBUF = 2
STREAMS = [(0, 128), (128, 128), (256, 128), (384, 16)]


def _transpose_block(in_ref, out_ref):
    out_ref[:, 0:D_MODEL] = jnp.transpose(in_ref[...], (1, 0))


def _transpose(table_t):
    vocab = table_t.shape[1]
    nblk = (vocab + W_T - 1) // W_T
    return pl.pallas_call(
        _transpose_block,
        grid=(nblk,),
        in_specs=[pl.BlockSpec((D_MODEL, W_T), lambda i: (0, i))],
        out_specs=pl.BlockSpec((W_T, DP), lambda i: (i, 0)),
        out_shape=jax.ShapeDtypeStruct((vocab, DP), jnp.float32),
    )(table_t)


@functools.lru_cache(maxsize=None)
def _gather(batch):
    info = plsc.get_sparse_core_info()
    nc, ns = info.num_cores, info.num_subcores
    nw = nc * ns
    rows_w = batch // nw
    idx_w = rows_w * T_SEQ
    n_chunks = rows_w // BROWS
    outer = n_chunks // NBUF
    ch_rows = BROWS * T_SEQ

    mesh = plsc.VectorSubcoreMesh(core_axis_name="c", subcore_axis_name="s")

    @functools.partial(
        pl.kernel,
        mesh=mesh,
        out_type=jax.ShapeDtypeStruct((batch * T_SEQ, D_MODEL), jnp.float32),
        scratch_types=[
            pltpu.VMEM((idx_w,), jnp.int32),
            pltpu.VMEM((NBUF, ch_rows, DP), jnp.float32),
            pltpu.SemaphoreType.DMA,
        ],
        compiler_params=pltpu.CompilerParams(use_tc_tiling_on_sc=False),
    )
    def gather_kernel(idx_hbm, tp_hbm, out_hbm, idx_v, rows_v, sem):
        wid = lax.axis_index("s") * nc + lax.axis_index("c")
        base_row = wid * idx_w
        pltpu.sync_copy(idx_hbm.at[pl.ds(base_row, idx_w)], idx_v)

        def fire(b, chunk):
            for off, sz in STREAMS:
                pltpu.async_copy(
                    tp_hbm.at[idx_v.at[pl.ds(chunk * ch_rows + off, sz)]],
                    rows_v.at[b, pl.ds(off, sz)], sem)

        def drain(b):
            for off, sz in STREAMS:
                pltpu.make_async_copy(
                    tp_hbm.at[idx_v.at[pl.ds(off, sz)]],
                    rows_v.at[b, pl.ds(off, sz)], sem).wait()

        def write(b, chunk):
            pltpu.sync_copy(
                rows_v.at[b, :, pl.ds(0, D_MODEL)],
                out_hbm.at[pl.ds(base_row + chunk * ch_rows, ch_rows)])

        fire(0, 0)

        def body(i, carry):
            c0 = i * NBUF
            fire(1, c0 + 1)
            drain(0)
            write(0, c0)

            @pl.when(i < outer - 1)
            def _():
                fire(0, c0 + 2)

            drain(1)
            write(1, c0 + 1)
            return carry

        lax.fori_loop(0, outer, body, 0)

    return gather_kernel


def kernel(x, table):
    b, t = x.shape
    idx = x.reshape(b * t).astype(jnp.int32)
    tp = _transpose(table.T)
    o = _gather(b)(idx, tp)
    return o.reshape(b, t, D_MODEL)
